# baseline (speedup 1.0000x reference)
import functools

import jax
import jax.numpy as jnp
from jax import lax
from jax.experimental import pallas as pl
from jax.experimental.pallas import tpu as pltpu
from jax.experimental.pallas import tpu_sc as plsc

_NUM_CORES = 2
_NUM_SUBCORES = 16
_NUM_WORKERS = _NUM_CORES * _NUM_SUBCORES
_CHUNK = 128
_NBUF = 4
_LANES = 128


@functools.lru_cache(maxsize=None)
def _make_transpose(n_rows, embed_dim):
    n_full = n_rows // _LANES
    tail = n_rows - n_full * _LANES
    n_blocks = n_full + (1 if tail else 0)
    n_main = n_full // _NUM_WORKERS
    n_rem = n_full - n_main * _NUM_WORKERS
    assert n_main % 2 == 0 and n_main >= 4
    mesh = plsc.VectorSubcoreMesh(core_axis_name="c", subcore_axis_name="s")

    def body(tab_t, tail_hbm, out_hbm, in_v0, in_v1, out_v0, out_v1, sem_i, sem_o):
        in_bufs = (in_v0, in_v1)
        out_bufs = (out_v0, out_v1)
        wid = lax.axis_index("s") * _NUM_CORES + lax.axis_index("c")
        iota = lax.iota(jnp.int32, 16)
        cvs = [iota + c0 for c0 in range(0, embed_dim, 16)]

        def in_start(step, slot):
            j = wid + step * _NUM_WORKERS
            for t in range(embed_dim // 8):
                pltpu.async_copy(
                    tab_t.at[pl.ds(t * 8, 8), pl.ds(j * _LANES, _LANES)],
                    in_bufs[slot].at[pl.ds(t * 8, 8)],
                    sem_i.at[slot],
                )

        def in_wait(slot):
            for t in range(embed_dim // 8):
                pltpu.make_async_copy(
                    tab_t.at[pl.ds(0, 8), pl.ds(0, _LANES)],
                    in_bufs[slot].at[pl.ds(t * 8, 8)],
                    sem_i.at[slot],
                ).wait()

        def out_start(step, slot):
            j = wid + step * _NUM_WORKERS
            pltpu.async_copy(out_bufs[slot], out_hbm.at[j], sem_o.at[slot])

        def out_wait(slot):
            pltpu.make_async_copy(
                out_bufs[slot], out_hbm.at[0], sem_o.at[slot]
            ).wait()

        def transpose_block(slot):
            def trow(r, carry):
                rv = lax.broadcast_in_dim(r, (16,), ())
                for ci, cv in enumerate(cvs):
                    out_bufs[slot][r, pl.ds(ci * 16, 16)] = plsc.load_gather(
                        in_bufs[slot], [cv, rv]
                    )
                return carry

            lax.fori_loop(0, _LANES, trow, 0)

        def step_fn(i, slot):
            in_wait(slot)

            @pl.when(i >= 2)
            def _():
                out_wait(slot)

            transpose_block(slot)
            out_start(i, slot)

            @pl.when(i + 2 < n_main)
            def _():
                in_start(i + 2, slot)

        in_start(0, 0)
        in_start(1, 1)

        def outer(k, carry):
            step_fn(2 * k, 0)
            step_fn(2 * k + 1, 1)
            return carry

        lax.fori_loop(0, n_main // 2, outer, 0)

        @pl.when(wid < n_rem)
        def _():
            in_start(n_main, 0)
            in_wait(0)
            out_wait(0)
            transpose_block(0)
            out_start(n_main, 0)

        if tail:

            @pl.when(wid == n_rem)
            def _():
                out_wait(1)
                pltpu.sync_copy(tail_hbm, out_bufs[1])
                pltpu.async_copy(out_bufs[1], out_hbm.at[n_full], sem_o.at[1])
                pltpu.make_async_copy(
                    out_bufs[1], out_hbm.at[0], sem_o.at[1]
                ).wait()

        out_wait(0)
        if tail:

            @pl.when(wid != n_rem)
            def _():
                out_wait(1)

        else:
            out_wait(1)

    return pl.kernel(
        body,
        out_type=jax.ShapeDtypeStruct((n_blocks, _LANES, _LANES), jnp.float32),
        mesh=mesh,
        scratch_types=[
            pltpu.VMEM((64, _LANES), jnp.float32),
            pltpu.VMEM((64, _LANES), jnp.float32),
            pltpu.VMEM((_LANES, _LANES), jnp.float32),
            pltpu.VMEM((_LANES, _LANES), jnp.float32),
            pltpu.SemaphoreType.DMA((2,)),
            pltpu.SemaphoreType.DMA((2,)),
        ],
        compiler_params=pltpu.CompilerParams(
            use_tc_tiling_on_sc=True, needs_layout_passes=False
        ),
    )


@functools.lru_cache(maxsize=None)
def _make_lookup(n_idx, n_pad):
    assert n_idx % (_NUM_WORKERS * _CHUNK) == 0
    n_chunks = n_idx // (_NUM_WORKERS * _CHUNK)
    assert n_chunks % _NBUF == 0 and n_chunks // _NBUF >= 2
    n_outer = n_chunks // _NBUF
    per_w = n_chunks * _CHUNK
    mesh = plsc.VectorSubcoreMesh(core_axis_name="c", subcore_axis_name="s")

    def body(table_hbm, idx_hbm, out_hbm, idx_v, rows_v, sem_g, sem_w):
        wid = lax.axis_index("s") * _NUM_CORES + lax.axis_index("c")
        base = wid * per_w
        pltpu.sync_copy(idx_hbm.at[wid], idx_v)

        def gather_start(g, slot):
            pltpu.async_copy(
                table_hbm.at[idx_v.at[g]], rows_v.at[slot], sem_g.at[slot]
            )

        def gather_wait(slot):
            pltpu.make_async_copy(
                table_hbm.at[idx_v.at[0]], rows_v.at[slot], sem_g.at[slot]
            ).wait()

        def write_start(g, slot):
            pltpu.async_copy(
                rows_v.at[slot],
                out_hbm.at[pl.ds(base + g * _CHUNK, _CHUNK)],
                sem_w.at[slot],
            )

        def write_wait(slot):
            pltpu.make_async_copy(
                rows_v.at[slot],
                out_hbm.at[pl.ds(base, _CHUNK)],
                sem_w.at[slot],
            ).wait()

        for b in range(_NBUF - 1):
            gather_start(b, b)

        def block(g0, first, last):
            for b in range(_NBUF):
                g = g0 + b
                gather_wait(b)
                write_start(g, b)
                if last and b > 0:
                    continue
                b2 = (b - 1) % _NBUF
                if not (first and b == 0):
                    write_wait(b2)
                gather_start(g + _NBUF - 1, b2)

        block(0, first=True, last=False)

        def outer(i, carry):
            block(i * _NBUF, first=False, last=False)
            return carry

        lax.fori_loop(1, n_outer - 1, outer, 0)
        block((n_outer - 1) * _NBUF, first=False, last=True)

        for b in range(_NBUF):
            write_wait(b)

    return pl.kernel(
        body,
        out_type=jax.ShapeDtypeStruct((n_idx, _LANES), jnp.float32),
        mesh=mesh,
        scratch_types=[
            pltpu.VMEM((n_chunks, _CHUNK), jnp.int32),
            pltpu.VMEM((_NBUF, _CHUNK, _LANES), jnp.float32),
            pltpu.SemaphoreType.DMA((_NBUF,)),
            pltpu.SemaphoreType.DMA((_NBUF,)),
        ],
        compiler_params=pltpu.CompilerParams(use_tc_tiling_on_sc=True),
    )


def kernel(token_ids, embedding):
    b, l = token_ids.shape
    n_idx = b * l
    n_rows, embed_dim = embedding.shape
    n_full = n_rows // _LANES
    tab_t = embedding.T
    tail_rows = jnp.pad(
        embedding[n_full * _LANES :],
        ((0, _LANES - (n_rows - n_full * _LANES)), (0, _LANES - embed_dim)),
    )
    blocks = _make_transpose(n_rows, embed_dim)(tab_t, tail_rows)
    table128 = blocks.reshape(-1, _LANES)
    idx = token_ids.reshape(_NUM_WORKERS, -1, _CHUNK).astype(jnp.int32)
    out = _make_lookup(n_idx, table128.shape[0])(table128, idx)
    out = lax.slice(out, (0, 0), (n_idx, embed_dim))
    return out.reshape(b, l, embed_dim)


# speedup vs baseline: 1.0009x; 1.0009x over previous
import functools

import jax
import jax.numpy as jnp
from jax import lax
from jax.experimental import pallas as pl
from jax.experimental.pallas import tpu as pltpu
from jax.experimental.pallas import tpu_sc as plsc

_NUM_CORES = 2
_NUM_SUBCORES = 16
_NUM_WORKERS = _NUM_CORES * _NUM_SUBCORES
_CHUNK = 128
_NBUF = 4
_LANES = 128


@functools.lru_cache(maxsize=None)
def _make_transpose(n_rows, embed_dim):
    n_full = n_rows // _LANES
    tail = n_rows - n_full * _LANES
    n_blocks = n_full + (1 if tail else 0)
    n_main = n_full // _NUM_WORKERS
    n_rem = n_full - n_main * _NUM_WORKERS
    assert n_main % 2 == 0 and n_main >= 4
    mesh = plsc.VectorSubcoreMesh(core_axis_name="c", subcore_axis_name="s")

    def body(tab_t, tail_hbm, out_hbm, in_v0, in_v1, out_v0, out_v1, sem_i, sem_o):
        in_bufs = (in_v0, in_v1)
        out_bufs = (out_v0, out_v1)
        wid = lax.axis_index("s") * _NUM_CORES + lax.axis_index("c")
        iota = lax.iota(jnp.int32, 16)
        cvs = [iota + c0 for c0 in range(0, embed_dim, 16)]

        def in_start(step, slot):
            j = wid + step * _NUM_WORKERS
            for t in range(embed_dim // 8):
                pltpu.async_copy(
                    tab_t.at[pl.ds(t * 8, 8), pl.ds(j * _LANES, _LANES)],
                    in_bufs[slot].at[pl.ds(t * 8, 8)],
                    sem_i.at[slot],
                )

        def in_wait(slot):
            for t in range(embed_dim // 8):
                pltpu.make_async_copy(
                    tab_t.at[pl.ds(0, 8), pl.ds(0, _LANES)],
                    in_bufs[slot].at[pl.ds(t * 8, 8)],
                    sem_i.at[slot],
                ).wait()

        def out_start(step, slot):
            j = wid + step * _NUM_WORKERS
            pltpu.async_copy(out_bufs[slot], out_hbm.at[j], sem_o.at[slot])

        def out_wait(slot):
            pltpu.make_async_copy(
                out_bufs[slot], out_hbm.at[0], sem_o.at[slot]
            ).wait()

        def transpose_block(slot):
            def trow(r, carry):
                rv = lax.broadcast_in_dim(r, (16,), ())
                for ci, cv in enumerate(cvs):
                    out_bufs[slot][r, pl.ds(ci * 16, 16)] = plsc.load_gather(
                        in_bufs[slot], [cv, rv]
                    )
                return carry

            lax.fori_loop(0, _LANES, trow, 0, unroll=8)

        def step_fn(i, slot):
            in_wait(slot)

            @pl.when(i >= 2)
            def _():
                out_wait(slot)

            transpose_block(slot)
            out_start(i, slot)

            @pl.when(i + 2 < n_main)
            def _():
                in_start(i + 2, slot)

        in_start(0, 0)
        in_start(1, 1)

        def outer(k, carry):
            step_fn(2 * k, 0)
            step_fn(2 * k + 1, 1)
            return carry

        lax.fori_loop(0, n_main // 2, outer, 0)

        @pl.when(wid < n_rem)
        def _():
            in_start(n_main, 0)
            in_wait(0)
            out_wait(0)
            transpose_block(0)
            out_start(n_main, 0)

        if tail:

            @pl.when(wid == n_rem)
            def _():
                out_wait(1)
                pltpu.sync_copy(tail_hbm, out_bufs[1])
                pltpu.async_copy(out_bufs[1], out_hbm.at[n_full], sem_o.at[1])
                pltpu.make_async_copy(
                    out_bufs[1], out_hbm.at[0], sem_o.at[1]
                ).wait()

        out_wait(0)
        if tail:

            @pl.when(wid != n_rem)
            def _():
                out_wait(1)

        else:
            out_wait(1)

    return pl.kernel(
        body,
        out_type=jax.ShapeDtypeStruct((n_blocks, _LANES, _LANES), jnp.float32),
        mesh=mesh,
        scratch_types=[
            pltpu.VMEM((64, _LANES), jnp.float32),
            pltpu.VMEM((64, _LANES), jnp.float32),
            pltpu.VMEM((_LANES, _LANES), jnp.float32),
            pltpu.VMEM((_LANES, _LANES), jnp.float32),
            pltpu.SemaphoreType.DMA((2,)),
            pltpu.SemaphoreType.DMA((2,)),
        ],
        compiler_params=pltpu.CompilerParams(
            use_tc_tiling_on_sc=True, needs_layout_passes=False
        ),
    )


@functools.lru_cache(maxsize=None)
def _make_lookup(n_idx, n_pad):
    assert n_idx % (_NUM_WORKERS * _CHUNK) == 0
    n_chunks = n_idx // (_NUM_WORKERS * _CHUNK)
    assert n_chunks % _NBUF == 0 and n_chunks // _NBUF >= 2
    n_outer = n_chunks // _NBUF
    per_w = n_chunks * _CHUNK
    mesh = plsc.VectorSubcoreMesh(core_axis_name="c", subcore_axis_name="s")

    def body(table_hbm, idx_hbm, out_hbm, idx_v, rows_v, sem_g, sem_w):
        wid = lax.axis_index("s") * _NUM_CORES + lax.axis_index("c")
        base = wid * per_w
        pltpu.sync_copy(idx_hbm.at[wid], idx_v)

        def gather_start(g, slot):
            pltpu.async_copy(
                table_hbm.at[idx_v.at[g]], rows_v.at[slot], sem_g.at[slot]
            )

        def gather_wait(slot):
            pltpu.make_async_copy(
                table_hbm.at[idx_v.at[0]], rows_v.at[slot], sem_g.at[slot]
            ).wait()

        def write_start(g, slot):
            pltpu.async_copy(
                rows_v.at[slot],
                out_hbm.at[pl.ds(base + g * _CHUNK, _CHUNK)],
                sem_w.at[slot],
            )

        def write_wait(slot):
            pltpu.make_async_copy(
                rows_v.at[slot],
                out_hbm.at[pl.ds(base, _CHUNK)],
                sem_w.at[slot],
            ).wait()

        for b in range(_NBUF - 1):
            gather_start(b, b)

        def block(g0, first, last):
            for b in range(_NBUF):
                g = g0 + b
                gather_wait(b)
                write_start(g, b)
                if last and b > 0:
                    continue
                b2 = (b - 1) % _NBUF
                if not (first and b == 0):
                    write_wait(b2)
                gather_start(g + _NBUF - 1, b2)

        block(0, first=True, last=False)

        def outer(i, carry):
            block(i * _NBUF, first=False, last=False)
            return carry

        lax.fori_loop(1, n_outer - 1, outer, 0)
        block((n_outer - 1) * _NBUF, first=False, last=True)

        for b in range(_NBUF):
            write_wait(b)

    return pl.kernel(
        body,
        out_type=jax.ShapeDtypeStruct((n_idx, _LANES), jnp.float32),
        mesh=mesh,
        scratch_types=[
            pltpu.VMEM((n_chunks, _CHUNK), jnp.int32),
            pltpu.VMEM((_NBUF, _CHUNK, _LANES), jnp.float32),
            pltpu.SemaphoreType.DMA((_NBUF,)),
            pltpu.SemaphoreType.DMA((_NBUF,)),
        ],
        compiler_params=pltpu.CompilerParams(use_tc_tiling_on_sc=True),
    )


def kernel(token_ids, embedding):
    b, l = token_ids.shape
    n_idx = b * l
    n_rows, embed_dim = embedding.shape
    n_full = n_rows // _LANES
    tab_t = embedding.T
    tail_rows = jnp.pad(
        embedding[n_full * _LANES :],
        ((0, _LANES - (n_rows - n_full * _LANES)), (0, _LANES - embed_dim)),
    )
    blocks = _make_transpose(n_rows, embed_dim)(tab_t, tail_rows)
    table128 = blocks.reshape(-1, _LANES)
    idx = token_ids.reshape(_NUM_WORKERS, -1, _CHUNK).astype(jnp.int32)
    out = _make_lookup(n_idx, table128.shape[0])(table128, idx)
    out = lax.slice(out, (0, 0), (n_idx, embed_dim))
    return out.reshape(b, l, embed_dim)


# speedup vs baseline: 1.9632x; 1.9614x over previous
import functools

import jax
import jax.numpy as jnp
from jax import lax
from jax.experimental import pallas as pl
from jax.experimental.pallas import tpu as pltpu
from jax.experimental.pallas import tpu_sc as plsc

_NUM_CORES = 2
_NUM_SUBCORES = 16
_NUM_WORKERS = _NUM_CORES * _NUM_SUBCORES
_CHUNK = 128
_NBUF = 4
_LANES = 128


@functools.lru_cache(maxsize=None)
def _make_lookup(n_idx, embed_dim):
    assert n_idx % (_NUM_WORKERS * _CHUNK) == 0
    n_chunks = n_idx // (_NUM_WORKERS * _CHUNK)
    assert n_chunks % _NBUF == 0 and n_chunks // _NBUF >= 2
    n_outer = n_chunks // _NBUF
    per_w = n_chunks * _CHUNK
    mesh = plsc.VectorSubcoreMesh(core_axis_name="c", subcore_axis_name="s")

    def body(table_hbm, idx_hbm, out_hbm, idx_v, rows_v, sem_g, sem_w):
        wid = lax.axis_index("s") * _NUM_CORES + lax.axis_index("c")
        base = wid * per_w
        pltpu.sync_copy(idx_hbm.at[wid], idx_v)

        def gather_start(g, slot):
            pltpu.async_copy(
                table_hbm.at[idx_v.at[g]], rows_v.at[slot], sem_g.at[slot]
            )

        def gather_wait(slot):
            pltpu.make_async_copy(
                table_hbm.at[idx_v.at[0]], rows_v.at[slot], sem_g.at[slot]
            ).wait()

        def write_start(g, slot):
            pltpu.async_copy(
                rows_v.at[slot],
                out_hbm.at[pl.ds(base + g * _CHUNK, _CHUNK)],
                sem_w.at[slot],
            )

        def write_wait(slot):
            pltpu.make_async_copy(
                rows_v.at[slot],
                out_hbm.at[pl.ds(base, _CHUNK)],
                sem_w.at[slot],
            ).wait()

        for b in range(_NBUF - 1):
            gather_start(b, b)

        def block(g0, first, last):
            for b in range(_NBUF):
                g = g0 + b
                gather_wait(b)
                write_start(g, b)
                if last and b > 0:
                    continue
                b2 = (b - 1) % _NBUF
                if not (first and b == 0):
                    write_wait(b2)
                gather_start(g + _NBUF - 1, b2)

        block(0, first=True, last=False)

        def outer(i, carry):
            block(i * _NBUF, first=False, last=False)
            return carry

        lax.fori_loop(1, n_outer - 1, outer, 0)
        block((n_outer - 1) * _NBUF, first=False, last=True)

        for b in range(_NBUF):
            write_wait(b)

    return pl.kernel(
        body,
        out_type=jax.ShapeDtypeStruct((n_idx, _LANES), jnp.float32),
        mesh=mesh,
        scratch_types=[
            pltpu.VMEM((n_chunks, _CHUNK), jnp.int32),
            pltpu.VMEM((_NBUF, _CHUNK, _LANES), jnp.float32),
            pltpu.SemaphoreType.DMA((_NBUF,)),
            pltpu.SemaphoreType.DMA((_NBUF,)),
        ],
        compiler_params=pltpu.CompilerParams(use_tc_tiling_on_sc=True),
    )


def kernel(token_ids, embedding):
    b, l = token_ids.shape
    n_idx = b * l
    embed_dim = embedding.shape[1]
    table_p = jnp.pad(embedding, ((0, 0), (0, _LANES - embed_dim)))
    idx = token_ids.reshape(_NUM_WORKERS, -1, _CHUNK).astype(jnp.int32)
    out = _make_lookup(n_idx, embed_dim)(table_p, idx)
    out = lax.slice(out, (0, 0), (n_idx, embed_dim))
    return out.reshape(b, l, embed_dim)
